# baseline (speedup 1.0000x reference)
import functools

import jax
import jax.numpy as jnp
from jax import lax
from jax.experimental import pallas as pl
from jax.experimental.pallas import tpu as pltpu
from jax.experimental.pallas import tpu_sc as plsc

N, D, E = 4096, 768, 8
B = 256
C = 2 * N + E * B
NB = C // B
NBP = 128
CH = 512
NW = 32
BPW = N // NW
CHD = 64
LANES = 16
WSW = 128

_SC_MESH = plsc.VectorSubcoreMesh(core_axis_name="c", subcore_axis_name="s")


def _gate_kernel(x_ref, Wg_ref, bg_ref, pos0_ref, pos1_ref, v1_ref, v2_ref,
                 emap_ref):
    x = x_ref[...]
    logits = lax.dot_general(
        x, Wg_ref[...], (((1,), (1,)), ((), ())),
        preferred_element_type=jnp.float32) + bg_ref[...][None, :]
    m = logits.max(axis=1, keepdims=True)
    ex = jnp.exp(logits - m)
    probs = ex / ex.sum(axis=1, keepdims=True)

    col = lax.broadcasted_iota(jnp.int32, probs.shape, 1)
    i1 = jnp.argmax(probs, axis=1)[:, None]
    v1 = jnp.max(probs, axis=1, keepdims=True)
    masked = jnp.where(col == i1, -jnp.inf, probs)
    i2 = jnp.argmax(masked, axis=1)[:, None]
    v2 = jnp.max(masked, axis=1, keepdims=True)
    oh0 = (col == i1).astype(jnp.float32)
    oh1 = (col == i2).astype(jnp.float32)

    r = lax.broadcasted_iota(jnp.int32, (CH, CH), 0)
    cidx = lax.broadcasted_iota(jnp.int32, (CH, CH), 1)
    T = (cidx < r).astype(jnp.float32)
    ec0, ec1 = [], []
    carry0 = jnp.zeros((1, E), jnp.float32)
    carry1 = jnp.zeros((1, E), jnp.float32)
    for c in range(N // CH):
        blk0 = oh0[c * CH:(c + 1) * CH]
        blk1 = oh1[c * CH:(c + 1) * CH]
        ec0.append(lax.dot_general(
            T, blk0, (((1,), (0,)), ((), ())),
            preferred_element_type=jnp.float32) + carry0)
        ec1.append(lax.dot_general(
            T, blk1, (((1,), (0,)), ((), ())),
            preferred_element_type=jnp.float32) + carry1)
        carry0 = carry0 + blk0.sum(axis=0, keepdims=True)
        carry1 = carry1 + blk1.sum(axis=0, keepdims=True)
    ec0 = jnp.concatenate(ec0, axis=0)
    ec1 = jnp.concatenate(ec1, axis=0)
    count0 = carry0
    cnt = carry0 + carry1
    pcount = jnp.ceil(cnt / B) * B
    eidx_r = lax.broadcasted_iota(jnp.int32, (E, E), 0)
    eidx_c = lax.broadcasted_iota(jnp.int32, (E, E), 1)
    U = (eidx_r < eidx_c).astype(jnp.float32)
    pbase = lax.dot_general(
        pcount, U, (((1,), (0,)), ((), ())),
        preferred_element_type=jnp.float32)

    pos0 = (oh0 * (pbase + ec0)).sum(axis=1, keepdims=True)
    pos1 = (oh1 * (pbase + count0 + ec1)).sum(axis=1, keepdims=True)
    pos0_ref[...] = jnp.broadcast_to(pos0.astype(jnp.int32), (N, 8))
    pos1_ref[...] = jnp.broadcast_to(pos1.astype(jnp.int32), (N, 8))
    v1_ref[...] = jnp.broadcast_to(v1, (N, WSW))
    v2_ref[...] = jnp.broadcast_to(v2, (N, WSW))

    pend = pbase + pcount
    jb = lax.broadcasted_iota(jnp.int32, (NBP, E), 0).astype(jnp.float32) * B
    emap = (jb >= jnp.broadcast_to(pend, (NBP, E))).astype(jnp.int32)
    emap = jnp.minimum(emap.sum(axis=1, keepdims=True), E - 1)
    emap_ref[...] = jnp.broadcast_to(emap, (NBP, 8))


def _gate(x, Wg, bg):
    return pl.pallas_call(
        _gate_kernel,
        out_shape=(
            jax.ShapeDtypeStruct((N, 8), jnp.int32),
            jax.ShapeDtypeStruct((N, 8), jnp.int32),
            jax.ShapeDtypeStruct((N, WSW), jnp.float32),
            jax.ShapeDtypeStruct((N, WSW), jnp.float32),
            jax.ShapeDtypeStruct((NBP, 8), jnp.int32),
        ),
    )(x, Wg, bg)


@functools.partial(
    pl.kernel,
    out_type=(jax.ShapeDtypeStruct((C, D), jnp.float32),
              jax.ShapeDtypeStruct((C, WSW), jnp.float32)),
    mesh=_SC_MESH,
    scratch_types=[pltpu.VMEM((BPW,), jnp.int32),
                   pltpu.VMEM((BPW,), jnp.int32),
                   pltpu.VMEM((BPW, D), jnp.float32),
                   pltpu.VMEM((BPW, WSW), jnp.float32),
                   pltpu.SemaphoreType.DMA],
)
def _dispatch(x_hbm, pos0_hbm, pos1_hbm, v1_hbm, v2_hbm, xs_hbm, ws_hbm,
              idx0_v, idx1_v, rows_v, w_v, sem):
    wid = lax.axis_index("s") * 2 + lax.axis_index("c")
    base = wid * BPW
    pltpu.sync_copy(pos0_hbm.at[pl.ds(base, BPW)], idx0_v)
    pltpu.sync_copy(pos1_hbm.at[pl.ds(base, BPW)], idx1_v)
    pltpu.sync_copy(x_hbm.at[pl.ds(base, BPW)], rows_v)
    pltpu.async_copy(rows_v, xs_hbm.at[idx0_v], sem).wait()
    pltpu.async_copy(rows_v, xs_hbm.at[idx1_v], sem).wait()
    pltpu.sync_copy(v1_hbm.at[pl.ds(base, BPW)], w_v)
    pltpu.async_copy(w_v, ws_hbm.at[idx0_v], sem).wait()
    pltpu.sync_copy(v2_hbm.at[pl.ds(base, BPW)], w_v)
    pltpu.async_copy(w_v, ws_hbm.at[idx1_v], sem).wait()


def _expert_kernel(emap_ref, xs_ref, W_ref, b_ref, ws_ref, ys_ref):
    h = lax.dot_general(
        xs_ref[...], W_ref[0], (((1,), (1,)), ((), ())),
        preferred_element_type=jnp.float32) + b_ref[0]
    ys_ref[...] = jnp.maximum(h, 0.0) * ws_ref[0][:, 0:1]


def _experts(emap, xs, W, b, ws3):
    grid_spec = pltpu.PrefetchScalarGridSpec(
        num_scalar_prefetch=1,
        grid=(NB,),
        in_specs=[
            pl.BlockSpec((B, D), lambda j, m: (j, 0)),
            pl.BlockSpec((1, D, D), lambda j, m: (m[j], 0, 0)),
            pl.BlockSpec((1, 1, D), lambda j, m: (m[j], 0, 0)),
            pl.BlockSpec((1, B, WSW), lambda j, m: (j, 0, 0)),
        ],
        out_specs=pl.BlockSpec((B, D), lambda j, m: (j, 0)),
    )
    return pl.pallas_call(
        _expert_kernel,
        grid_spec=grid_spec,
        out_shape=jax.ShapeDtypeStruct((C, D), jnp.float32),
    )(emap, xs, W, b.reshape(E, 1, D), ws3)


@functools.partial(
    pl.kernel,
    out_type=jax.ShapeDtypeStruct((N, D), jnp.float32),
    mesh=_SC_MESH,
    scratch_types=[pltpu.VMEM((CHD,), jnp.int32),
                   pltpu.VMEM((CHD,), jnp.int32),
                   pltpu.VMEM((CHD, D), jnp.float32),
                   pltpu.VMEM((CHD, D), jnp.float32),
                   pltpu.SemaphoreType.DMA],
)
def _combine(ys_hbm, pos0_hbm, pos1_hbm, out_hbm, idx0_v, idx1_v, r0_v, r1_v,
             sem):
    wid = lax.axis_index("s") * 2 + lax.axis_index("c")
    for chunk in range(BPW // CHD):
        base = wid * BPW + chunk * CHD
        pltpu.sync_copy(pos0_hbm.at[pl.ds(base, CHD)], idx0_v)
        pltpu.sync_copy(pos1_hbm.at[pl.ds(base, CHD)], idx1_v)
        pltpu.async_copy(ys_hbm.at[idx0_v], r0_v, sem).wait()
        pltpu.async_copy(ys_hbm.at[idx1_v], r1_v, sem).wait()

        def _row(i, _):
            for j in range(D // LANES):
                plsc.addupdate(r0_v.at[i, pl.ds(j * LANES, LANES)],
                               r1_v[i, pl.ds(j * LANES, LANES)])
            return 0

        lax.fori_loop(0, CHD, _row, 0)
        pltpu.sync_copy(r0_v, out_hbm.at[pl.ds(base, CHD)])


@jax.jit
def kernel(x, W, b, Wg, bg):
    pos0b, pos1b, v1b, v2b, emapb = _gate(x, Wg, bg)
    pos0, pos1 = pos0b[:, 0], pos1b[:, 0]
    emap = emapb[:, 0]
    xs, ws = _dispatch(x, pos0, pos1, v1b, v2b)
    ys = _experts(emap, xs, W, b, ws.reshape(NB, B, WSW))
    return _combine(ys, pos0, pos1)


# speedup vs baseline: 2.3378x; 2.3378x over previous
import jax
import jax.numpy as jnp
from jax import lax
from jax.experimental import pallas as pl


def _moe_block(x_ref, Wf_ref, bf_ref, Wg_ref, bg_ref, out_ref):
    xb = x_ref[...]
    T, D = xb.shape
    E = Wg_ref.shape[0]
    logits = lax.dot_general(
        xb, Wg_ref[...], (((1,), (1,)), ((), ())),
        preferred_element_type=jnp.float32) + bg_ref[...][None, :]
    m = logits.max(axis=1, keepdims=True)
    ex = jnp.exp(logits - m)
    probs = ex / ex.sum(axis=1, keepdims=True)

    col = lax.broadcasted_iota(jnp.int32, probs.shape, 1)
    i1 = jnp.argmax(probs, axis=1)[:, None]
    v1 = jnp.max(probs, axis=1, keepdims=True)
    masked = jnp.where(col == i1, -jnp.inf, probs)
    i2 = jnp.argmax(masked, axis=1)[:, None]
    v2 = jnp.max(masked, axis=1, keepdims=True)
    gate = jnp.where(col == i1, v1, jnp.where(col == i2, v2, 0.0))

    h = lax.dot_general(
        xb, Wf_ref[...], (((1,), (1,)), ((), ())),
        preferred_element_type=jnp.float32) + bf_ref[...]
    h = jnp.maximum(h, 0.0)
    acc = jnp.zeros((T, D), jnp.float32)
    for e in range(E):
        acc = acc + gate[:, e][:, None] * h[:, e * D:(e + 1) * D]
    out_ref[...] = acc


@jax.jit
def kernel(x, W, b, Wg, bg):
    N, D = x.shape
    E = W.shape[0]
    Wf = W.reshape(E * D, D)
    bf = b.reshape(1, E * D)
    T = 512
    grid = (N // T,)
    return pl.pallas_call(
        _moe_block,
        grid=grid,
        in_specs=[
            pl.BlockSpec((T, D), lambda i: (i, 0)),
            pl.BlockSpec((E * D, D), lambda i: (0, 0)),
            pl.BlockSpec((1, E * D), lambda i: (0, 0)),
            pl.BlockSpec((E, D), lambda i: (0, 0)),
            pl.BlockSpec((E,), lambda i: (0,)),
        ],
        out_specs=pl.BlockSpec((T, D), lambda i: (i, 0)),
        out_shape=jax.ShapeDtypeStruct((N, D), x.dtype),
    )(x, Wf, bf, Wg, bg)
